# baseline (speedup 1.0000x reference)
import functools

import jax
import jax.numpy as jnp
from jax import lax
from jax.experimental import pallas as pl
from jax.experimental.pallas import tpu as pltpu
from jax.experimental.pallas import tpu_sc as plsc

_LOW_D = 10.0
_HIGH_D = 30.0
_HIGH_R = 0.01
_LOW_R = 0.005


_OOB = 1 << 28


def _precompute_body(dur, ret, dirn, start, stop, bat, mar, fs3_o, fp3_o,
                     fsw_o, fpw_o, w_o, *, M, BM):
    d = dur[...]
    r = ret[...]
    di = dirn[...]
    dscore = jnp.where(d > _HIGH_D, 1.0, jnp.where(d < _LOW_D, 0.5, 0.75))
    rscore = jnp.where(r > _HIGH_R, 1.0, jnp.where(r < _LOW_R, 0.0, 0.75))
    score = (dscore * rscore).astype(jnp.float32)
    side = (r < _LOW_R) | ((d > _HIGH_D) & (r < _HIGH_R))
    not_side = jnp.logical_not(side)
    up = not_side & (di > 0)
    down = not_side & (di < 0)
    cat = jnp.where(up, 0, jnp.where(side, 1, jnp.where(down, 2, 3)))
    w = jnp.where(cat != 3, score, 0.0)
    w_o[...] = w
    col = bat[...] * M + mar[...]
    fs = start[...] * BM + col
    fp = stop[...] * BM + col
    incat = cat < 3
    oob_s = _OOB + (fs & 8191)
    oob_p = _OOB + (fp & 8191)
    fs3_o[...] = jnp.where(incat, fs * 3 + cat, oob_s)
    fp3_o[...] = jnp.where(incat, fp * 3 + cat, oob_p)
    haw = w != 0.0
    fsw_o[...] = jnp.where(haw, fs, oob_s)
    fpw_o[...] = jnp.where(haw, fp, oob_p)


def _precompute(duration, returns, direction, start_at, stop_at, batch_idx,
                market, M, BM):
    n = duration.shape[0]
    assert n % 128 == 0
    rows = n // 128
    br = rows
    for cand in (512, 256, 128, 64, 8):
        if rows % cand == 0:
            br = cand
            break
    g = rows // br
    shp = (rows, 128)
    ins = [x.reshape(shp) for x in (duration, returns, direction, start_at,
                                    stop_at, batch_idx, market)]
    spec = pl.BlockSpec((br, 128), lambda i: (i, 0))
    outs = pl.pallas_call(
        functools.partial(_precompute_body, M=M, BM=BM),
        grid=(g,),
        in_specs=[spec] * 7,
        out_specs=[spec] * 5,
        out_shape=[
            jax.ShapeDtypeStruct(shp, jnp.int32),
            jax.ShapeDtypeStruct(shp, jnp.int32),
            jax.ShapeDtypeStruct(shp, jnp.int32),
            jax.ShapeDtypeStruct(shp, jnp.int32),
            jax.ShapeDtypeStruct(shp, jnp.float32),
        ],
    )(*ins)
    return tuple(o.reshape(n) for o in outs)


def _sc_scatter(fs3, fp3, fsw, fpw, w, T, BM, n_e=1024, trash=8192):
    npts = T * BM
    quar = npts // 4
    half = npts // 2
    cat_words = quar * 3
    acc_words = cat_words + trash
    npad = fs3.shape[0]
    ev_per_tile = npad // 16
    nblk = ev_per_tile // n_e
    assert ev_per_tile % n_e == 0 and nblk % 2 == 0 and n_e % 16 == 0
    zchunk = 4096
    zw_cat = cat_words // 16
    zw_w = half // 16
    assert zw_cat % zchunk == 0 and zw_w % zchunk == 0
    mesh = plsc.VectorSubcoreMesh(core_axis_name="c", subcore_axis_name="s")

    def body(fs3_hbm, fp3_hbm, fsw_hbm, fpw_hbm, w_hbm, cats_hbm, w_out_hbm,
             fs_v0, fs_v1, fp_v0, fp_v1, w_v0, w_v1,
             sidx0, sidx1, sval0, sval1, cval, zero_v,
             sem_st0, sem_st1, sem_sc0, sem_sc1, acc_sh):
        core = lax.axis_index("c")
        tid = lax.axis_index("s")
        stage = ((fs_v0, fp_v0, w_v0, sidx0, sval0, sem_st0, sem_sc0),
                 (fs_v1, fp_v1, w_v1, sidx1, sval1, sem_st1, sem_sc1))

        def zinit(i, _):
            zero_v[pl.ds(i * 16, 16)] = jnp.zeros((16,), jnp.float32)
            return _

        lax.fori_loop(0, zchunk // 16, zinit, None)

        def cinit(i, _):
            cval[pl.ds(i * 16, 16)] = jnp.full((16,), 1.0, jnp.float32)
            cval[pl.ds(n_e + i * 16, 16)] = jnp.full((16,), -1.0, jnp.float32)
            return _

        lax.fori_loop(0, n_e // 16, cinit, None)

        def run_scan(is_cat, piece, out_hbm, out_off, zw):
            def zblk(j, _):
                pltpu.sync_copy(zero_v,
                                acc_sh.at[pl.ds(tid * zw + j * zchunk, zchunk)])
                return _

            lax.fori_loop(0, zw // zchunk, zblk, None)
            plsc.subcore_barrier()

            piece_off = piece * (quar * 3 if is_cat else half)
            bound = jnp.uint32(quar * 3 if is_cat else half)

            def st_descs(bi, p):
                base = tid * ev_per_tile + bi * n_e
                fs_v, fp_v, w_v, _, _, sem_st, _ = stage[p]
                s_hbm, p_hbm = (fs3_hbm, fp3_hbm) if is_cat else (fsw_hbm,
                                                                  fpw_hbm)
                d = [pltpu.make_async_copy(s_hbm.at[pl.ds(base, n_e)], fs_v,
                                           sem_st),
                     pltpu.make_async_copy(p_hbm.at[pl.ds(base, n_e)], fp_v,
                                           sem_st)]
                if not is_cat:
                    d.append(pltpu.make_async_copy(
                        w_hbm.at[pl.ds(base, n_e)], w_v, sem_st))
                return d

            def sc_desc(p):
                _, _, _, sidx, sval, _, sem_sc = stage[p]
                src = cval if is_cat else sval
                return pltpu.make_async_copy(src, acc_sh.at[sidx], sem_sc)

            def fire_scat(p):
                _, _, _, sidx, sval, _, sem_sc = stage[p]
                src = cval if is_cat else sval
                pltpu.async_copy(src, acc_sh.at[sidx], sem_sc, add=True)

            def compute(p):
                fs_v, fp_v, w_v, sidx, sval, _, _ = stage[p]

                def lane(i, _):
                    s = fs_v[pl.ds(i * 16, 16)] - piece_off
                    q = fp_v[pl.ds(i * 16, 16)] - piece_off
                    trs = cat_words + (s & (trash - 1))
                    trq = cat_words + (q & (trash - 1))
                    oks = plsc.bitcast(s, jnp.uint32) < bound
                    okq = plsc.bitcast(q, jnp.uint32) < bound
                    sidx[pl.ds(i * 16, 16)] = jnp.where(oks, s, trs)
                    sidx[pl.ds(n_e + i * 16, 16)] = jnp.where(okq, q, trq)
                    if not is_cat:
                        v = w_v[pl.ds(i * 16, 16)]
                        sval[pl.ds(i * 16, 16)] = v
                        sval[pl.ds(n_e + i * 16, 16)] = -v
                    return _

                lax.fori_loop(0, n_e // 16, lane, None, unroll=4)

            for dd in st_descs(0, 0):
                dd.start()

            def body2(j, _):
                b0 = 2 * j
                for dd in st_descs(b0, 0):
                    dd.wait()

                for dd in st_descs(b0 + 1, 1):
                    dd.start()

                @pl.when(j > 0)
                def _():
                    sc_desc(0).wait()

                compute(0)
                fire_scat(0)
                for dd in st_descs(b0 + 1, 1):
                    dd.wait()

                @pl.when(j + 1 < nblk // 2)
                def _():
                    for dd in st_descs(b0 + 2, 0):
                        dd.start()

                @pl.when(j > 0)
                def _():
                    sc_desc(1).wait()

                compute(1)
                fire_scat(1)
                return _

            lax.fori_loop(0, nblk // 2, body2, None)
            sc_desc(0).wait()
            sc_desc(1).wait()
            plsc.subcore_barrier()
            pltpu.sync_copy(acc_sh.at[pl.ds(tid * zw, zw)],
                            out_hbm.at[pl.ds(out_off + tid * zw, zw)])
            plsc.subcore_barrier()

        for jq in range(2):
            qq = core * 2 + jq
            run_scan(True, qq, cats_hbm, qq * cat_words, zw_cat)
        run_scan(False, core, w_out_hbm, core * half, zw_w)

    run = pl.kernel(
        body,
        out_type=(jax.ShapeDtypeStruct((npts * 3,), jnp.float32),
                  jax.ShapeDtypeStruct((npts,), jnp.float32)),
        mesh=mesh,
        scratch_types=[
            pltpu.VMEM((n_e,), jnp.int32),
            pltpu.VMEM((n_e,), jnp.int32),
            pltpu.VMEM((n_e,), jnp.int32),
            pltpu.VMEM((n_e,), jnp.int32),
            pltpu.VMEM((n_e,), jnp.float32),
            pltpu.VMEM((n_e,), jnp.float32),
            pltpu.VMEM((2 * n_e,), jnp.int32),
            pltpu.VMEM((2 * n_e,), jnp.int32),
            pltpu.VMEM((2 * n_e,), jnp.float32),
            pltpu.VMEM((2 * n_e,), jnp.float32),
            pltpu.VMEM((2 * n_e,), jnp.float32),
            pltpu.VMEM((zchunk,), jnp.float32),
            pltpu.SemaphoreType.DMA,
            pltpu.SemaphoreType.DMA,
            pltpu.SemaphoreType.DMA,
            pltpu.SemaphoreType.DMA,
            pltpu.VMEM_SHARED((acc_words,), jnp.float32),
        ],
    )
    return run(fs3, fp3, fsw, fpw, w)


def _cumsum_body(acc_ref, out_ref, carry_ref, *, BT):
    @pl.when(pl.program_id(1) == 0)
    def _():
        carry_ref[...] = jnp.zeros_like(carry_ref)

    blk = acc_ref[0]
    row = lax.broadcasted_iota(jnp.int32, (BT, BT), 0)
    col = lax.broadcasted_iota(jnp.int32, (BT, BT), 1)
    tri = (row >= col).astype(jnp.float32)
    cum = jax.lax.dot(tri, blk, preferred_element_type=jnp.float32)
    out_ref[0] = cum + carry_ref[...]
    carry_ref[...] = carry_ref[...] + cum[BT - 1:BT, :]


def _cumsum(acc, T, C, BT=512):
    acc2 = acc.reshape(1, T, C)
    spec = pl.BlockSpec((1, BT, C), lambda c, t: (c, t, 0))
    out = pl.pallas_call(
        functools.partial(_cumsum_body, BT=BT),
        grid=(1, T // BT),
        in_specs=[spec],
        out_specs=spec,
        out_shape=jax.ShapeDtypeStruct((1, T, C), jnp.float32),
        scratch_shapes=[pltpu.VMEM((1, C), jnp.float32)],
    )(acc2)
    return out.reshape(T, C)


def kernel(duration, returns, direction, start_at, stop_at, batch_idx, market):
    n = duration.shape[0]
    T = 4096
    B = 8
    M = 64
    BM = B * M
    n_e = 1024
    grp = 16 * 2 * n_e
    npad = ((n + grp - 1) // grp) * grp
    pad = npad - n
    if pad:
        spread = (jnp.arange(pad, dtype=jnp.int32) * 7) % T
        duration = jnp.concatenate([duration, jnp.zeros((pad,), jnp.float32)])
        returns = jnp.concatenate([returns, jnp.ones((pad,), jnp.float32)])
        direction = jnp.concatenate([direction, jnp.zeros((pad,), jnp.float32)])
        start_at = jnp.concatenate([start_at, spread])
        stop_at = jnp.concatenate([stop_at, spread])
        batch_idx = jnp.concatenate([batch_idx, jnp.zeros((pad,), jnp.int32)])
        market = jnp.concatenate([market, jnp.zeros((pad,), jnp.int32)])
    fs3, fp3, fsw, fpw, w = _precompute(duration, returns, direction,
                                        start_at, stop_at, batch_idx, market,
                                        M, BM)
    acc_cats, acc_w = _sc_scatter(fs3, fp3, fsw, fpw, w, T, BM, n_e=n_e)
    cats = _cumsum(acc_cats, T, BM * 3).reshape(T, B, M, 3)
    mask = _cumsum(acc_w, T, BM).reshape(T, B, M)
    return cats, mask


# speedup vs baseline: 1.0059x; 1.0059x over previous
import functools

import jax
import jax.numpy as jnp
from jax import lax
from jax.experimental import pallas as pl
from jax.experimental.pallas import tpu as pltpu
from jax.experimental.pallas import tpu_sc as plsc

_LOW_D = 10.0
_HIGH_D = 30.0
_HIGH_R = 0.01
_LOW_R = 0.005


_OOB = 1 << 28


def _precompute_body(dur, ret, dirn, start, stop, bat, mar, fs3_o, fp3_o,
                     fsw_o, fpw_o, w_o, *, M, BM):
    d = dur[...]
    r = ret[...]
    di = dirn[...]
    dscore = jnp.where(d > _HIGH_D, 1.0, jnp.where(d < _LOW_D, 0.5, 0.75))
    rscore = jnp.where(r > _HIGH_R, 1.0, jnp.where(r < _LOW_R, 0.0, 0.75))
    score = (dscore * rscore).astype(jnp.float32)
    side = (r < _LOW_R) | ((d > _HIGH_D) & (r < _HIGH_R))
    not_side = jnp.logical_not(side)
    up = not_side & (di > 0)
    down = not_side & (di < 0)
    cat = jnp.where(up, 0, jnp.where(side, 1, jnp.where(down, 2, 3)))
    w = jnp.where(cat != 3, score, 0.0)
    w_o[...] = w
    col = bat[...] * M + mar[...]
    fs = start[...] * BM + col
    fp = stop[...] * BM + col
    incat = cat < 3
    oob_s = _OOB + (fs & 8191)
    oob_p = _OOB + (fp & 8191)
    fs3_o[...] = jnp.where(incat, fs * 3 + cat, oob_s)
    fp3_o[...] = jnp.where(incat, fp * 3 + cat, oob_p)
    haw = w != 0.0
    fsw_o[...] = jnp.where(haw, fs, oob_s)
    fpw_o[...] = jnp.where(haw, fp, oob_p)


def _precompute(duration, returns, direction, start_at, stop_at, batch_idx,
                market, M, BM):
    n = duration.shape[0]
    assert n % 128 == 0
    rows = n // 128
    br = rows
    for cand in (512, 256, 128, 64, 8):
        if rows % cand == 0:
            br = cand
            break
    g = rows // br
    shp = (rows, 128)
    ins = [x.reshape(shp) for x in (duration, returns, direction, start_at,
                                    stop_at, batch_idx, market)]
    spec = pl.BlockSpec((br, 128), lambda i: (i, 0))
    outs = pl.pallas_call(
        functools.partial(_precompute_body, M=M, BM=BM),
        grid=(g,),
        in_specs=[spec] * 7,
        out_specs=[spec] * 5,
        out_shape=[
            jax.ShapeDtypeStruct(shp, jnp.int32),
            jax.ShapeDtypeStruct(shp, jnp.int32),
            jax.ShapeDtypeStruct(shp, jnp.int32),
            jax.ShapeDtypeStruct(shp, jnp.int32),
            jax.ShapeDtypeStruct(shp, jnp.float32),
        ],
    )(*ins)
    return tuple(o.reshape(n) for o in outs)


def _sc_scatter(fs3, fp3, fsw, fpw, w, T, BM, n_e=1024, trash=8192):
    npts = T * BM
    quar = npts // 4
    half = npts // 2
    cat_words = quar * 3
    acc_words = cat_words + trash
    npad = fs3.shape[0]
    ev_per_tile = npad // 16
    nblk = ev_per_tile // n_e
    assert ev_per_tile % n_e == 0 and nblk % 2 == 0 and n_e % 16 == 0
    zchunk = 4096
    zw_cat = cat_words // 16
    zw_w = half // 16
    assert zw_cat % zchunk == 0 and zw_w % zchunk == 0
    mesh = plsc.VectorSubcoreMesh(core_axis_name="c", subcore_axis_name="s")

    def body(fs3_hbm, fp3_hbm, fsw_hbm, fpw_hbm, w_hbm, cats_hbm, w_out_hbm,
             fs_v0, fs_v1, fp_v0, fp_v1, w_v0, w_v1,
             sidx0, sidx1, sval0, sval1, cval, zero_v,
             sem_st0, sem_st1, sem_sc0, sem_sc1, acc_sh):
        core = lax.axis_index("c")
        tid = lax.axis_index("s")
        stage = ((fs_v0, fp_v0, w_v0, sidx0, sval0, sem_st0, sem_sc0),
                 (fs_v1, fp_v1, w_v1, sidx1, sval1, sem_st1, sem_sc1))

        def zinit(i, _):
            zero_v[pl.ds(i * 16, 16)] = jnp.zeros((16,), jnp.float32)
            return _

        lax.fori_loop(0, zchunk // 16, zinit, None)

        def cinit(i, _):
            cval[pl.ds(i * 16, 16)] = jnp.full((16,), 1.0, jnp.float32)
            cval[pl.ds(n_e + i * 16, 16)] = jnp.full((16,), -1.0, jnp.float32)
            return _

        lax.fori_loop(0, n_e // 16, cinit, None)

        def run_scan(is_cat, piece, out_hbm, out_off, zw):
            def zblk(j, _):
                pltpu.sync_copy(zero_v,
                                acc_sh.at[pl.ds(tid * zw + j * zchunk, zchunk)])
                return _

            lax.fori_loop(0, zw // zchunk, zblk, None)
            plsc.subcore_barrier()

            piece_off = piece * (quar * 3 if is_cat else half)
            bound = jnp.uint32(quar * 3 if is_cat else half)

            def st_descs(bi, p):
                base = tid * ev_per_tile + bi * n_e
                fs_v, fp_v, w_v, _, _, sem_st, _ = stage[p]
                s_hbm, p_hbm = (fs3_hbm, fp3_hbm) if is_cat else (fsw_hbm,
                                                                  fpw_hbm)
                d = [pltpu.make_async_copy(s_hbm.at[pl.ds(base, n_e)], fs_v,
                                           sem_st),
                     pltpu.make_async_copy(p_hbm.at[pl.ds(base, n_e)], fp_v,
                                           sem_st)]
                if not is_cat:
                    d.append(pltpu.make_async_copy(
                        w_hbm.at[pl.ds(base, n_e)], w_v, sem_st))
                return d

            def sc_desc(p):
                _, _, _, sidx, sval, _, sem_sc = stage[p]
                src = cval if is_cat else sval
                return pltpu.make_async_copy(src, acc_sh.at[sidx], sem_sc)

            def fire_scat(p):
                _, _, _, sidx, sval, _, sem_sc = stage[p]
                src = cval if is_cat else sval
                pltpu.async_copy(src, acc_sh.at[sidx], sem_sc, add=True)

            def compute(p):
                fs_v, fp_v, w_v, sidx, sval, _, _ = stage[p]

                def lane(i, _):
                    s = fs_v[pl.ds(i * 16, 16)] - piece_off
                    q = fp_v[pl.ds(i * 16, 16)] - piece_off
                    trs = cat_words + (s & (trash - 1))
                    trq = cat_words + (q & (trash - 1))
                    oks = plsc.bitcast(s, jnp.uint32) < bound
                    okq = plsc.bitcast(q, jnp.uint32) < bound
                    sidx[pl.ds(i * 16, 16)] = jnp.where(oks, s, trs)
                    sidx[pl.ds(n_e + i * 16, 16)] = jnp.where(okq, q, trq)
                    if not is_cat:
                        v = w_v[pl.ds(i * 16, 16)]
                        sval[pl.ds(i * 16, 16)] = v
                        sval[pl.ds(n_e + i * 16, 16)] = -v
                    return _

                lax.fori_loop(0, n_e // 16, lane, None)

            for dd in st_descs(0, 0):
                dd.start()

            def body2(j, _):
                b0 = 2 * j
                for dd in st_descs(b0, 0):
                    dd.wait()

                for dd in st_descs(b0 + 1, 1):
                    dd.start()

                @pl.when(j > 0)
                def _():
                    sc_desc(0).wait()

                compute(0)
                fire_scat(0)
                for dd in st_descs(b0 + 1, 1):
                    dd.wait()

                @pl.when(j + 1 < nblk // 2)
                def _():
                    for dd in st_descs(b0 + 2, 0):
                        dd.start()

                @pl.when(j > 0)
                def _():
                    sc_desc(1).wait()

                compute(1)
                fire_scat(1)
                return _

            lax.fori_loop(0, nblk // 2, body2, None)
            sc_desc(0).wait()
            sc_desc(1).wait()
            plsc.subcore_barrier()
            pltpu.sync_copy(acc_sh.at[pl.ds(tid * zw, zw)],
                            out_hbm.at[pl.ds(out_off + tid * zw, zw)])
            plsc.subcore_barrier()

        for jq in range(2):
            qq = core * 2 + jq
            run_scan(True, qq, cats_hbm, qq * cat_words, zw_cat)
        run_scan(False, core, w_out_hbm, core * half, zw_w)

    run = pl.kernel(
        body,
        out_type=(jax.ShapeDtypeStruct((npts * 3,), jnp.float32),
                  jax.ShapeDtypeStruct((npts,), jnp.float32)),
        mesh=mesh,
        scratch_types=[
            pltpu.VMEM((n_e,), jnp.int32),
            pltpu.VMEM((n_e,), jnp.int32),
            pltpu.VMEM((n_e,), jnp.int32),
            pltpu.VMEM((n_e,), jnp.int32),
            pltpu.VMEM((n_e,), jnp.float32),
            pltpu.VMEM((n_e,), jnp.float32),
            pltpu.VMEM((2 * n_e,), jnp.int32),
            pltpu.VMEM((2 * n_e,), jnp.int32),
            pltpu.VMEM((2 * n_e,), jnp.float32),
            pltpu.VMEM((2 * n_e,), jnp.float32),
            pltpu.VMEM((2 * n_e,), jnp.float32),
            pltpu.VMEM((zchunk,), jnp.float32),
            pltpu.SemaphoreType.DMA,
            pltpu.SemaphoreType.DMA,
            pltpu.SemaphoreType.DMA,
            pltpu.SemaphoreType.DMA,
            pltpu.VMEM_SHARED((acc_words,), jnp.float32),
        ],
    )
    return run(fs3, fp3, fsw, fpw, w)


def _cumsum_body(acc_ref, out_ref, carry_ref, *, BT):
    @pl.when(pl.program_id(1) == 0)
    def _():
        carry_ref[...] = jnp.zeros_like(carry_ref)

    blk = acc_ref[0]
    row = lax.broadcasted_iota(jnp.int32, (BT, BT), 0)
    col = lax.broadcasted_iota(jnp.int32, (BT, BT), 1)
    tri = (row >= col).astype(jnp.float32)
    cum = jax.lax.dot(tri, blk, preferred_element_type=jnp.float32)
    out_ref[0] = cum + carry_ref[...]
    carry_ref[...] = carry_ref[...] + cum[BT - 1:BT, :]


def _cumsum(acc, T, C, BT=512):
    acc2 = acc.reshape(1, T, C)
    spec = pl.BlockSpec((1, BT, C), lambda c, t: (c, t, 0))
    out = pl.pallas_call(
        functools.partial(_cumsum_body, BT=BT),
        grid=(1, T // BT),
        in_specs=[spec],
        out_specs=spec,
        out_shape=jax.ShapeDtypeStruct((1, T, C), jnp.float32),
        scratch_shapes=[pltpu.VMEM((1, C), jnp.float32)],
    )(acc2)
    return out.reshape(T, C)


def kernel(duration, returns, direction, start_at, stop_at, batch_idx, market):
    n = duration.shape[0]
    T = 4096
    B = 8
    M = 64
    BM = B * M
    n_e = 1024
    grp = 16 * 2 * n_e
    npad = ((n + grp - 1) // grp) * grp
    pad = npad - n
    if pad:
        spread = (jnp.arange(pad, dtype=jnp.int32) * 7) % T
        duration = jnp.concatenate([duration, jnp.zeros((pad,), jnp.float32)])
        returns = jnp.concatenate([returns, jnp.ones((pad,), jnp.float32)])
        direction = jnp.concatenate([direction, jnp.zeros((pad,), jnp.float32)])
        start_at = jnp.concatenate([start_at, spread])
        stop_at = jnp.concatenate([stop_at, spread])
        batch_idx = jnp.concatenate([batch_idx, jnp.zeros((pad,), jnp.int32)])
        market = jnp.concatenate([market, jnp.zeros((pad,), jnp.int32)])
    fs3, fp3, fsw, fpw, w = _precompute(duration, returns, direction,
                                        start_at, stop_at, batch_idx, market,
                                        M, BM)
    acc_cats, acc_w = _sc_scatter(fs3, fp3, fsw, fpw, w, T, BM, n_e=n_e)
    cats = _cumsum(acc_cats, T, BM * 3).reshape(T, B, M, 3)
    mask = _cumsum(acc_w, T, BM).reshape(T, B, M)
    return cats, mask


# speedup vs baseline: 1.1451x; 1.1383x over previous
import functools

import jax
import jax.numpy as jnp
from jax import lax
from jax.experimental import pallas as pl
from jax.experimental.pallas import tpu as pltpu
from jax.experimental.pallas import tpu_sc as plsc

_LOW_D = 10.0
_HIGH_D = 30.0
_HIGH_R = 0.01
_LOW_R = 0.005


_OOB = 1 << 28


def _precompute_body(dur, ret, dirn, start, stop, bat, mar, fs3_o, fp3_o,
                     fsw_o, fpw_o, w_o, *, M, BM):
    d = dur[...]
    r = ret[...]
    di = dirn[...]
    dscore = jnp.where(d > _HIGH_D, 1.0, jnp.where(d < _LOW_D, 0.5, 0.75))
    rscore = jnp.where(r > _HIGH_R, 1.0, jnp.where(r < _LOW_R, 0.0, 0.75))
    score = (dscore * rscore).astype(jnp.float32)
    side = (r < _LOW_R) | ((d > _HIGH_D) & (r < _HIGH_R))
    not_side = jnp.logical_not(side)
    up = not_side & (di > 0)
    down = not_side & (di < 0)
    cat = jnp.where(up, 0, jnp.where(side, 1, jnp.where(down, 2, 3)))
    w = jnp.where(cat != 3, score, 0.0)
    w_o[...] = w
    col = bat[...] * M + mar[...]
    fs = start[...] * BM + col
    fp = stop[...] * BM + col
    incat = cat < 3
    oob_s = _OOB + (fs & 8191)
    oob_p = _OOB + (fp & 8191)
    fs3_o[...] = jnp.where(incat, fs * 3 + cat, oob_s)
    fp3_o[...] = jnp.where(incat, fp * 3 + cat, oob_p)
    haw = w != 0.0
    fsw_o[...] = jnp.where(haw, fs, oob_s)
    fpw_o[...] = jnp.where(haw, fp, oob_p)


def _precompute(duration, returns, direction, start_at, stop_at, batch_idx,
                market, M, BM):
    n = duration.shape[0]
    assert n % 128 == 0
    rows = n // 128
    br = rows
    for cand in (512, 256, 128, 64, 8):
        if rows % cand == 0:
            br = cand
            break
    g = rows // br
    shp = (rows, 128)
    ins = [x.reshape(shp) for x in (duration, returns, direction, start_at,
                                    stop_at, batch_idx, market)]
    spec = pl.BlockSpec((br, 128), lambda i: (i, 0))
    outs = pl.pallas_call(
        functools.partial(_precompute_body, M=M, BM=BM),
        grid=(g,),
        in_specs=[spec] * 7,
        out_specs=[spec] * 5,
        out_shape=[
            jax.ShapeDtypeStruct(shp, jnp.int32),
            jax.ShapeDtypeStruct(shp, jnp.int32),
            jax.ShapeDtypeStruct(shp, jnp.int32),
            jax.ShapeDtypeStruct(shp, jnp.int32),
            jax.ShapeDtypeStruct(shp, jnp.float32),
        ],
    )(*ins)
    return tuple(o.reshape(n) for o in outs)


def _sc_scatter(fs3, fp3, fsw, fpw, w, T, BM, n_e=1024, trash=8192):
    npts = T * BM
    quar = npts // 4
    half = npts // 2
    cat_words = quar * 3
    acc_words = cat_words + trash
    npad = fs3.shape[0]
    ev_per_tile = npad // 16
    nblk = ev_per_tile // n_e
    assert ev_per_tile % n_e == 0 and nblk % 2 == 0 and n_e % 16 == 0
    zchunk = 4096
    zw_cat = cat_words // 16
    zw_w = half // 16
    assert zw_cat % zchunk == 0 and zw_w % zchunk == 0
    mesh = plsc.VectorSubcoreMesh(core_axis_name="c", subcore_axis_name="s")

    def body(fs3_hbm, fp3_hbm, fsw_hbm, fpw_hbm, w_hbm, cats_hbm, w_out_hbm,
             fs_v0, fs_v1, fp_v0, fp_v1, w_v0, w_v1,
             sidx0, sidx1, sval0, sval1, cval, zero_v,
             sem_st0, sem_st1, sem_sc0, sem_sc1, acc_sh):
        core = lax.axis_index("c")
        tid = lax.axis_index("s")
        stage = ((fs_v0, fp_v0, w_v0, sidx0, sval0, sem_st0, sem_sc0),
                 (fs_v1, fp_v1, w_v1, sidx1, sval1, sem_st1, sem_sc1))

        def zinit(i, _):
            zero_v[pl.ds(i * 16, 16)] = jnp.zeros((16,), jnp.float32)
            return _

        lax.fori_loop(0, zchunk // 16, zinit, None)

        def cinit(i, _):
            cval[pl.ds(i * 16, 16)] = jnp.full((16,), 1.0, jnp.float32)
            cval[pl.ds(n_e + i * 16, 16)] = jnp.full((16,), -1.0, jnp.float32)
            return _

        lax.fori_loop(0, n_e // 16, cinit, None)

        def run_scan(is_cat, piece, out_hbm, out_off, zw):
            def zblk(j, _):
                pltpu.sync_copy(zero_v,
                                acc_sh.at[pl.ds(tid * zw + j * zchunk, zchunk)])
                return _

            lax.fori_loop(0, zw // zchunk, zblk, None)
            plsc.subcore_barrier()

            piece_off = piece * (quar * 3 if is_cat else half)
            bound = jnp.uint32(quar * 3 if is_cat else half)

            def st_descs(bi, p):
                base = tid * ev_per_tile + bi * n_e
                fs_v, fp_v, w_v, _, _, sem_st, _ = stage[p]
                s_hbm, p_hbm = (fs3_hbm, fp3_hbm) if is_cat else (fsw_hbm,
                                                                  fpw_hbm)
                d = [pltpu.make_async_copy(s_hbm.at[pl.ds(base, n_e)], fs_v,
                                           sem_st),
                     pltpu.make_async_copy(p_hbm.at[pl.ds(base, n_e)], fp_v,
                                           sem_st)]
                if not is_cat:
                    d.append(pltpu.make_async_copy(
                        w_hbm.at[pl.ds(base, n_e)], w_v, sem_st))
                return d

            def sc_desc(p):
                _, _, _, sidx, sval, _, sem_sc = stage[p]
                src = cval if is_cat else sval
                return pltpu.make_async_copy(src, acc_sh.at[sidx], sem_sc)

            def fire_scat(p):
                _, _, _, sidx, sval, _, sem_sc = stage[p]
                src = cval if is_cat else sval
                pltpu.async_copy(src, acc_sh.at[sidx], sem_sc, add=True)

            def compute(p):
                fs_v, fp_v, w_v, sidx, sval, _, _ = stage[p]

                def lane(i, _):
                    s = fs_v[pl.ds(i * 16, 16)] - piece_off
                    q = fp_v[pl.ds(i * 16, 16)] - piece_off
                    trs = cat_words + (s & (trash - 1))
                    trq = cat_words + (q & (trash - 1))
                    oks = plsc.bitcast(s, jnp.uint32) < bound
                    okq = plsc.bitcast(q, jnp.uint32) < bound
                    sidx[pl.ds(i * 16, 16)] = jnp.where(oks, s, trs)
                    sidx[pl.ds(n_e + i * 16, 16)] = jnp.where(okq, q, trq)
                    if not is_cat:
                        v = w_v[pl.ds(i * 16, 16)]
                        sval[pl.ds(i * 16, 16)] = v
                        sval[pl.ds(n_e + i * 16, 16)] = -v
                    return _

                lax.fori_loop(0, n_e // 16, lane, None)

            for dd in st_descs(0, 0):
                dd.start()

            def body2(j, _):
                b0 = 2 * j
                for dd in st_descs(b0, 0):
                    dd.wait()

                for dd in st_descs(b0 + 1, 1):
                    dd.start()

                @pl.when(j > 0)
                def _():
                    sc_desc(0).wait()

                compute(0)
                fire_scat(0)
                for dd in st_descs(b0 + 1, 1):
                    dd.wait()

                @pl.when(j + 1 < nblk // 2)
                def _():
                    for dd in st_descs(b0 + 2, 0):
                        dd.start()

                @pl.when(j > 0)
                def _():
                    sc_desc(1).wait()

                compute(1)
                fire_scat(1)
                return _

            lax.fori_loop(0, nblk // 2, body2, None)
            sc_desc(0).wait()
            sc_desc(1).wait()
            plsc.subcore_barrier()
            pltpu.sync_copy(acc_sh.at[pl.ds(tid * zw, zw)],
                            out_hbm.at[pl.ds(out_off + tid * zw, zw)])
            plsc.subcore_barrier()

        for jq in range(2):
            qq = core * 2 + jq
            run_scan(True, qq, cats_hbm, qq * cat_words, zw_cat)
        run_scan(False, core, w_out_hbm, core * half, zw_w)

    run = pl.kernel(
        body,
        out_type=(jax.ShapeDtypeStruct((npts * 3,), jnp.float32),
                  jax.ShapeDtypeStruct((npts,), jnp.float32)),
        mesh=mesh,
        scratch_types=[
            pltpu.VMEM((n_e,), jnp.int32),
            pltpu.VMEM((n_e,), jnp.int32),
            pltpu.VMEM((n_e,), jnp.int32),
            pltpu.VMEM((n_e,), jnp.int32),
            pltpu.VMEM((n_e,), jnp.float32),
            pltpu.VMEM((n_e,), jnp.float32),
            pltpu.VMEM((2 * n_e,), jnp.int32),
            pltpu.VMEM((2 * n_e,), jnp.int32),
            pltpu.VMEM((2 * n_e,), jnp.float32),
            pltpu.VMEM((2 * n_e,), jnp.float32),
            pltpu.VMEM((2 * n_e,), jnp.float32),
            pltpu.VMEM((zchunk,), jnp.float32),
            pltpu.SemaphoreType.DMA,
            pltpu.SemaphoreType.DMA,
            pltpu.SemaphoreType.DMA,
            pltpu.SemaphoreType.DMA,
            pltpu.VMEM_SHARED((acc_words,), jnp.float32),
        ],
    )
    return run(fs3, fp3, fsw, fpw, w)


def _cumsum_body(acc_ref, out_ref, carry_ref, *, BT):
    @pl.when(pl.program_id(1) == 0)
    def _():
        carry_ref[...] = jnp.zeros_like(carry_ref)

    blk = acc_ref[0]
    row = lax.broadcasted_iota(jnp.int32, (BT, BT), 0)
    col = lax.broadcasted_iota(jnp.int32, (BT, BT), 1)
    tri = (row >= col).astype(jnp.float32)
    cum = jax.lax.dot(tri, blk, preferred_element_type=jnp.float32)
    out_ref[0] = cum + carry_ref[...]
    carry_ref[...] = carry_ref[...] + cum[BT - 1:BT, :]


def _cumsum(acc, T, C, BT=512):
    acc2 = acc.reshape(1, T, C)
    spec = pl.BlockSpec((1, BT, C), lambda c, t: (c, t, 0))
    out = pl.pallas_call(
        functools.partial(_cumsum_body, BT=BT),
        grid=(1, T // BT),
        in_specs=[spec],
        out_specs=spec,
        out_shape=jax.ShapeDtypeStruct((1, T, C), jnp.float32),
        scratch_shapes=[pltpu.VMEM((1, C), jnp.float32)],
    )(acc2)
    return out.reshape(T, C)


def kernel(duration, returns, direction, start_at, stop_at, batch_idx, market):
    n = duration.shape[0]
    T = 4096
    B = 8
    M = 64
    BM = B * M
    n_e = 1024
    grp = 16 * 2 * n_e
    npad = ((n + grp - 1) // grp) * grp
    pad = npad - n
    if pad:
        ar = jnp.arange(pad, dtype=jnp.int32)
        duration = jnp.concatenate([duration, jnp.zeros((pad,), jnp.float32)])
        returns = jnp.concatenate([returns, jnp.ones((pad,), jnp.float32)])
        direction = jnp.concatenate([direction, jnp.zeros((pad,), jnp.float32)])
        start_at = jnp.concatenate([start_at, (ar >> 9) % T])
        stop_at = jnp.concatenate([stop_at, (ar >> 9) % T])
        batch_idx = jnp.concatenate([batch_idx, (ar >> 6) & 7])
        market = jnp.concatenate([market, ar & 63])
    fs3, fp3, fsw, fpw, w = _precompute(duration, returns, direction,
                                        start_at, stop_at, batch_idx, market,
                                        M, BM)
    acc_cats, acc_w = _sc_scatter(fs3, fp3, fsw, fpw, w, T, BM, n_e=n_e)
    cats = _cumsum(acc_cats, T, BM * 3).reshape(T, B, M, 3)
    mask = _cumsum(acc_w, T, BM).reshape(T, B, M)
    return cats, mask


# speedup vs baseline: 1.2357x; 1.0792x over previous
import functools

import jax
import jax.numpy as jnp
from jax import lax
from jax.experimental import pallas as pl
from jax.experimental.pallas import tpu as pltpu
from jax.experimental.pallas import tpu_sc as plsc

_LOW_D = 10.0
_HIGH_D = 30.0
_HIGH_R = 0.01
_LOW_R = 0.005


_OOB = 1 << 28


def _precompute_body(dur, ret, dirn, start, stop, bat, mar, fs3_o, fp3_o,
                     fsw_o, fpw_o, w_o, *, M, BM, n, br):
    d = dur[...]
    r = ret[...]
    di = dirn[...]
    shp = d.shape
    eid = (pl.program_id(0) * br * 128
           + lax.broadcasted_iota(jnp.int32, shp, 0) * 128
           + lax.broadcasted_iota(jnp.int32, shp, 1))
    valid = eid < n
    dscore = jnp.where(d > _HIGH_D, 1.0, jnp.where(d < _LOW_D, 0.5, 0.75))
    rscore = jnp.where(r > _HIGH_R, 1.0, jnp.where(r < _LOW_R, 0.0, 0.75))
    score = (dscore * rscore).astype(jnp.float32)
    side = (r < _LOW_R) | ((d > _HIGH_D) & (r < _HIGH_R))
    not_side = jnp.logical_not(side)
    up = not_side & (di > 0)
    down = not_side & (di < 0)
    cat = jnp.where(up, 0, jnp.where(side, 1, jnp.where(down, 2, 3)))
    w = jnp.where(valid & (cat != 3), score, 0.0)
    w_o[...] = w
    col = bat[...] * M + mar[...]
    fs = start[...] * BM + col
    fp = stop[...] * BM + col
    incat = valid & (cat < 3)
    oob = _OOB + (eid & 8191)
    fs3_o[...] = jnp.where(incat, fs * 3 + cat, oob)
    fp3_o[...] = jnp.where(incat, fp * 3 + cat, oob)
    haw = w != 0.0
    fsw_o[...] = jnp.where(haw, fs, oob)
    fpw_o[...] = jnp.where(haw, fp, oob)


def _precompute(duration, returns, direction, start_at, stop_at, batch_idx,
                market, M, BM, npad):
    n = duration.shape[0]
    assert n % 128 == 0 and npad % 128 == 0
    rows_in = n // 128
    rows_out = npad // 128
    br = rows_out
    for cand in (496, 512, 256, 128, 64, 8):
        if rows_out % cand == 0:
            br = cand
            break
    g = rows_out // br
    ins = [x.reshape(rows_in, 128)
           for x in (duration, returns, direction, start_at, stop_at,
                     batch_idx, market)]
    spec = pl.BlockSpec((br, 128), lambda i: (i, 0))
    oshp = (rows_out, 128)
    outs = pl.pallas_call(
        functools.partial(_precompute_body, M=M, BM=BM, n=n, br=br),
        grid=(g,),
        in_specs=[spec] * 7,
        out_specs=[spec] * 5,
        out_shape=[
            jax.ShapeDtypeStruct(oshp, jnp.int32),
            jax.ShapeDtypeStruct(oshp, jnp.int32),
            jax.ShapeDtypeStruct(oshp, jnp.int32),
            jax.ShapeDtypeStruct(oshp, jnp.int32),
            jax.ShapeDtypeStruct(oshp, jnp.float32),
        ],
    )(*ins)
    return tuple(o.reshape(npad) for o in outs)


def _sc_scatter(fs3, fp3, fsw, fpw, w, T, BM, n_e=1024, trash=8192):
    npts = T * BM
    quar = npts // 4
    half = npts // 2
    cat_words = quar * 3
    acc_words = cat_words + trash
    npad = fs3.shape[0]
    ev_per_tile = npad // 16
    nblk = ev_per_tile // n_e
    assert ev_per_tile % n_e == 0 and nblk % 2 == 0 and n_e % 16 == 0
    zchunk = 4096
    zw_cat = cat_words // 16
    zw_w = half // 16
    assert zw_cat % zchunk == 0 and zw_w % zchunk == 0
    mesh = plsc.VectorSubcoreMesh(core_axis_name="c", subcore_axis_name="s")

    def body(fs3_hbm, fp3_hbm, fsw_hbm, fpw_hbm, w_hbm, cats_hbm, w_out_hbm,
             fs_v0, fs_v1, fp_v0, fp_v1, w_v0, w_v1,
             sidx0, sidx1, sval0, sval1, cval, zero_v,
             sem_st0, sem_st1, sem_sc0, sem_sc1, acc_sh):
        core = lax.axis_index("c")
        tid = lax.axis_index("s")
        stage = ((fs_v0, fp_v0, w_v0, sidx0, sval0, sem_st0, sem_sc0),
                 (fs_v1, fp_v1, w_v1, sidx1, sval1, sem_st1, sem_sc1))

        def zinit(i, _):
            zero_v[pl.ds(i * 16, 16)] = jnp.zeros((16,), jnp.float32)
            return _

        lax.fori_loop(0, zchunk // 16, zinit, None)

        def cinit(i, _):
            cval[pl.ds(i * 16, 16)] = jnp.full((16,), 1.0, jnp.float32)
            cval[pl.ds(n_e + i * 16, 16)] = jnp.full((16,), -1.0, jnp.float32)
            return _

        lax.fori_loop(0, n_e // 16, cinit, None)

        def run_scan(is_cat, piece, out_hbm, out_off, zw):
            def zblk(j, _):
                pltpu.sync_copy(zero_v,
                                acc_sh.at[pl.ds(tid * zw + j * zchunk, zchunk)])
                return _

            lax.fori_loop(0, zw // zchunk, zblk, None)
            plsc.subcore_barrier()

            piece_off = piece * (quar * 3 if is_cat else half)
            bound = jnp.uint32(quar * 3 if is_cat else half)

            def st_descs(bi, p):
                base = tid * ev_per_tile + bi * n_e
                fs_v, fp_v, w_v, _, _, sem_st, _ = stage[p]
                s_hbm, p_hbm = (fs3_hbm, fp3_hbm) if is_cat else (fsw_hbm,
                                                                  fpw_hbm)
                d = [pltpu.make_async_copy(s_hbm.at[pl.ds(base, n_e)], fs_v,
                                           sem_st),
                     pltpu.make_async_copy(p_hbm.at[pl.ds(base, n_e)], fp_v,
                                           sem_st)]
                if not is_cat:
                    d.append(pltpu.make_async_copy(
                        w_hbm.at[pl.ds(base, n_e)], w_v, sem_st))
                return d

            def sc_desc(p):
                _, _, _, sidx, sval, _, sem_sc = stage[p]
                src = cval if is_cat else sval
                return pltpu.make_async_copy(src, acc_sh.at[sidx], sem_sc)

            def fire_scat(p):
                _, _, _, sidx, sval, _, sem_sc = stage[p]
                src = cval if is_cat else sval
                pltpu.async_copy(src, acc_sh.at[sidx], sem_sc, add=True)

            def compute(p):
                fs_v, fp_v, w_v, sidx, sval, _, _ = stage[p]

                def lane(i, _):
                    s = fs_v[pl.ds(i * 16, 16)] - piece_off
                    q = fp_v[pl.ds(i * 16, 16)] - piece_off
                    trs = cat_words + (s & (trash - 1))
                    trq = cat_words + (q & (trash - 1))
                    oks = plsc.bitcast(s, jnp.uint32) < bound
                    okq = plsc.bitcast(q, jnp.uint32) < bound
                    sidx[pl.ds(i * 16, 16)] = jnp.where(oks, s, trs)
                    sidx[pl.ds(n_e + i * 16, 16)] = jnp.where(okq, q, trq)
                    if not is_cat:
                        v = w_v[pl.ds(i * 16, 16)]
                        sval[pl.ds(i * 16, 16)] = v
                        sval[pl.ds(n_e + i * 16, 16)] = -v
                    return _

                lax.fori_loop(0, n_e // 16, lane, None)

            for dd in st_descs(0, 0):
                dd.start()

            def body2(j, _):
                b0 = 2 * j
                for dd in st_descs(b0, 0):
                    dd.wait()

                for dd in st_descs(b0 + 1, 1):
                    dd.start()

                @pl.when(j > 0)
                def _():
                    sc_desc(0).wait()

                compute(0)
                fire_scat(0)
                for dd in st_descs(b0 + 1, 1):
                    dd.wait()

                @pl.when(j + 1 < nblk // 2)
                def _():
                    for dd in st_descs(b0 + 2, 0):
                        dd.start()

                @pl.when(j > 0)
                def _():
                    sc_desc(1).wait()

                compute(1)
                fire_scat(1)
                return _

            lax.fori_loop(0, nblk // 2, body2, None)
            sc_desc(0).wait()
            sc_desc(1).wait()
            plsc.subcore_barrier()
            pltpu.sync_copy(acc_sh.at[pl.ds(tid * zw, zw)],
                            out_hbm.at[pl.ds(out_off + tid * zw, zw)])
            plsc.subcore_barrier()

        for jq in range(2):
            qq = core * 2 + jq
            run_scan(True, qq, cats_hbm, qq * cat_words, zw_cat)
        run_scan(False, core, w_out_hbm, core * half, zw_w)

    run = pl.kernel(
        body,
        out_type=(jax.ShapeDtypeStruct((npts * 3,), jnp.float32),
                  jax.ShapeDtypeStruct((npts,), jnp.float32)),
        mesh=mesh,
        scratch_types=[
            pltpu.VMEM((n_e,), jnp.int32),
            pltpu.VMEM((n_e,), jnp.int32),
            pltpu.VMEM((n_e,), jnp.int32),
            pltpu.VMEM((n_e,), jnp.int32),
            pltpu.VMEM((n_e,), jnp.float32),
            pltpu.VMEM((n_e,), jnp.float32),
            pltpu.VMEM((2 * n_e,), jnp.int32),
            pltpu.VMEM((2 * n_e,), jnp.int32),
            pltpu.VMEM((2 * n_e,), jnp.float32),
            pltpu.VMEM((2 * n_e,), jnp.float32),
            pltpu.VMEM((2 * n_e,), jnp.float32),
            pltpu.VMEM((zchunk,), jnp.float32),
            pltpu.SemaphoreType.DMA,
            pltpu.SemaphoreType.DMA,
            pltpu.SemaphoreType.DMA,
            pltpu.SemaphoreType.DMA,
            pltpu.VMEM_SHARED((acc_words,), jnp.float32),
        ],
    )
    return run(fs3, fp3, fsw, fpw, w)


def _cumsum_body(acc_ref, out_ref, carry_ref, *, BT):
    @pl.when(pl.program_id(1) == 0)
    def _():
        carry_ref[...] = jnp.zeros_like(carry_ref)

    blk = acc_ref[0]
    row = lax.broadcasted_iota(jnp.int32, (BT, BT), 0)
    col = lax.broadcasted_iota(jnp.int32, (BT, BT), 1)
    tri = (row >= col).astype(jnp.float32)
    cum = jax.lax.dot(tri, blk, preferred_element_type=jnp.float32)
    out_ref[0] = cum + carry_ref[...]
    carry_ref[...] = carry_ref[...] + cum[BT - 1:BT, :]


def _cumsum(acc, T, C, BT=512):
    acc2 = acc.reshape(1, T, C)
    spec = pl.BlockSpec((1, BT, C), lambda c, t: (c, t, 0))
    out = pl.pallas_call(
        functools.partial(_cumsum_body, BT=BT),
        grid=(1, T // BT),
        in_specs=[spec],
        out_specs=spec,
        out_shape=jax.ShapeDtypeStruct((1, T, C), jnp.float32),
        scratch_shapes=[pltpu.VMEM((1, C), jnp.float32)],
    )(acc2)
    return out.reshape(T, C)


def kernel(duration, returns, direction, start_at, stop_at, batch_idx, market):
    n = duration.shape[0]
    T = 4096
    B = 8
    M = 64
    BM = B * M
    n_e = 1024
    grp = 16 * 2 * n_e
    npad = ((n + grp - 1) // grp) * grp
    fs3, fp3, fsw, fpw, w = _precompute(duration, returns, direction,
                                        start_at, stop_at, batch_idx, market,
                                        M, BM, npad)
    acc_cats, acc_w = _sc_scatter(fs3, fp3, fsw, fpw, w, T, BM, n_e=n_e)
    cats = _cumsum(acc_cats, T, BM * 3).reshape(T, B, M, 3)
    mask = _cumsum(acc_w, T, BM).reshape(T, B, M)
    return cats, mask
